# speedup vs baseline: 2.5944x; 2.5944x over previous
import functools

import jax
import jax.numpy as jnp
from jax import lax
from jax.experimental import pallas as pl
from jax.experimental.pallas import tpu as pltpu
from jax.experimental.pallas import tpu_sc as plsc

E = 64
D = 768
H = 128
N = 2048
BT = 128
NB = N // BT
G = NB + E
BG = 256

SC_NW = 32
BPW = N // SC_NW


def _gating(x, w_gate):
    def body(x_ref, wg_ref, idx_ref, xs_ref):
        xv = x_ref[...]
        logits = jnp.dot(xv, wg_ref[...], preferred_element_type=jnp.float32)
        m = jnp.max(logits, axis=1, keepdims=True)
        s = jnp.sum(jnp.exp(logits - m), axis=1, keepdims=True)
        iota = lax.broadcasted_iota(jnp.int32, logits.shape, 1)
        cand = jnp.where(logits == m, iota, E)
        idx_ref[...] = jnp.min(cand, axis=1).astype(jnp.int32)
        xs_ref[...] = xv * (1.0 / s)

    return pl.pallas_call(
        body,
        grid=(N // BG,),
        in_specs=[
            pl.BlockSpec((BG, D), lambda i: (i, 0)),
            pl.BlockSpec((D, E), lambda i: (0, 0)),
        ],
        out_specs=[
            pl.BlockSpec((BG,), lambda i: (i,)),
            pl.BlockSpec((BG, D), lambda i: (i, 0)),
        ],
        out_shape=[
            jax.ShapeDtypeStruct((N,), jnp.int32),
            jax.ShapeDtypeStruct((N, D), jnp.float32),
        ],
    )(x, w_gate)


def _sc_gather(table, idx):
    mesh = plsc.VectorSubcoreMesh(core_axis_name="c", subcore_axis_name="s")

    @functools.partial(
        pl.kernel,
        mesh=mesh,
        out_type=jax.ShapeDtypeStruct((N, D), jnp.float32),
        scratch_types=[
            pltpu.VMEM((BPW,), jnp.int32),
            pltpu.VMEM((BPW, D), jnp.float32),
            pltpu.SemaphoreType.DMA,
        ],
    )
    def k(table_hbm, idx_hbm, out_hbm, idx_v, rows_v, sem):
        wid = lax.axis_index("s") * 2 + lax.axis_index("c")
        base = wid * BPW
        pltpu.sync_copy(idx_hbm.at[pl.ds(base, BPW)], idx_v)
        pltpu.async_copy(table_hbm.at[idx_v], rows_v, sem).wait()
        pltpu.sync_copy(rows_v, out_hbm.at[pl.ds(base, BPW)])

    return k(table, idx)


def _gmm(bid, eid, lo_s, hi_s, x_sorted, w1, w2):
    def body(bid_ref, eid_ref, lo_ref, hi_ref, x_ref, w1_ref, w2_ref, out_ref):
        g = pl.program_id(0)
        is_first = jnp.logical_or(
            g == 0, bid_ref[jnp.maximum(g - 1, 0)] != bid_ref[g]
        )

        @pl.when(is_first)
        def _():
            out_ref[...] = jnp.zeros_like(out_ref)

        base = bid_ref[g] * BT
        rows = base + lax.broadcasted_iota(jnp.int32, (BT, 1), 0)
        mask = jnp.logical_and(rows >= lo_ref[g], rows < hi_ref[g])
        xb = jnp.where(mask, x_ref[...], 0.0)
        h = jnp.dot(xb, w1_ref[0], preferred_element_type=jnp.float32)
        y = jnp.dot(h, w2_ref[0], preferred_element_type=jnp.float32)
        out_ref[...] += y

    grid_spec = pltpu.PrefetchScalarGridSpec(
        num_scalar_prefetch=4,
        grid=(G,),
        in_specs=[
            pl.BlockSpec((BT, D), lambda g, b, e, l, h: (b[g], 0)),
            pl.BlockSpec((1, D, H), lambda g, b, e, l, h: (e[g], 0, 0)),
            pl.BlockSpec((1, H, D), lambda g, b, e, l, h: (e[g], 0, 0)),
        ],
        out_specs=pl.BlockSpec((BT, D), lambda g, b, e, l, h: (b[g], 0)),
    )
    return pl.pallas_call(
        body,
        grid_spec=grid_spec,
        out_shape=jax.ShapeDtypeStruct((N, D), jnp.float32),
        compiler_params=pltpu.CompilerParams(
            dimension_semantics=("arbitrary",)
        ),
    )(bid, eid, lo_s, hi_s, x_sorted, w1, w2)


def _schedule(idx):
    counts = jnp.bincount(idx, length=E).astype(jnp.int32)
    off = (jnp.cumsum(counts) - counts).astype(jnp.int32)
    blk_lo = jnp.arange(NB, dtype=jnp.int32)[:, None] * BT
    lo = jnp.maximum(off[None, :], blk_lo)
    hi = jnp.minimum((off + counts)[None, :], blk_lo + BT)
    valid = (lo < hi).reshape(-1)
    rank = jnp.arange(NB * E, dtype=jnp.int32)
    key = jnp.where(valid, rank, jnp.int32(NB * E))
    sel = jnp.argsort(key)[:G]
    sval = valid[sel]
    bid = jnp.where(sval, sel // E, NB - 1).astype(jnp.int32)
    eid = jnp.where(sval, sel % E, 0).astype(jnp.int32)
    lo_s = jnp.where(sval, lo.reshape(-1)[sel], 0).astype(jnp.int32)
    hi_s = jnp.where(sval, hi.reshape(-1)[sel], 0).astype(jnp.int32)
    return bid, eid, lo_s, hi_s


def kernel(x, w_gate, w1, w2):
    idx, x_scaled = _gating(x, w_gate)
    order = jnp.argsort(idx, stable=True).astype(jnp.int32)
    bid, eid, lo_s, hi_s = _schedule(idx)
    x_sorted = _sc_gather(x_scaled, order)
    y_sorted = _gmm(bid, eid, lo_s, hi_s, x_sorted, w1, w2)
    inv = jnp.zeros((N,), jnp.int32).at[order].set(
        jnp.arange(N, dtype=jnp.int32)
    )
    return _sc_gather(y_sorted, inv)
